# baseline (speedup 1.0000x reference)
import functools

import jax
import jax.numpy as jnp
from jax import lax
from jax.experimental import pallas as pl
from jax.experimental.pallas import tpu as pltpu
from jax.experimental.pallas import tpu_sc as plsc

B = 4096
E = 128
H = 256
NC = 2
NS = 16
NW = NC * NS
BPW = B // NW
CH = 32


def _gather_body(user_table, item_table, uid, iid, out,
                 uidx, iidx, urows, irows, isem, usem, vsem, wsem):
    wid = lax.axis_index("s") * NC + lax.axis_index("c")
    base = wid * BPW
    ui = pltpu.async_copy(uid.at[pl.ds(base, BPW)], uidx, isem)
    vi = pltpu.async_copy(iid.at[pl.ds(base, BPW)], iidx, isem)
    ui.wait()
    vi.wait()
    gathers = []
    for c in range(0, BPW, CH):
        gathers.append((pltpu.async_copy(user_table.at[uidx.at[pl.ds(c, CH)]],
                                         urows.at[pl.ds(c, CH)], usem),
                        urows.at[pl.ds(c, CH)], base + c))
        gathers.append((pltpu.async_copy(item_table.at[iidx.at[pl.ds(c, CH)]],
                                         irows.at[pl.ds(c, CH)], vsem),
                        irows.at[pl.ds(c, CH)], B + base + c))
    writes = []
    for cp, rows, dst in gathers:
        cp.wait()
        writes.append(pltpu.async_copy(rows, out.at[pl.ds(dst, CH)], wsem))
    for w in writes:
        w.wait()


def _sc_gather(user_table, item_table, uid, iid):
    mesh = plsc.VectorSubcoreMesh(core_axis_name="c", subcore_axis_name="s")
    f = functools.partial(
        pl.kernel,
        mesh=mesh,
        out_type=jax.ShapeDtypeStruct((2 * B, E), jnp.float32),
        scratch_types=[
            pltpu.VMEM((BPW,), jnp.int32),
            pltpu.VMEM((BPW,), jnp.int32),
            pltpu.VMEM((BPW, E), jnp.float32),
            pltpu.VMEM((BPW, E), jnp.float32),
            pltpu.SemaphoreType.DMA,
            pltpu.SemaphoreType.DMA,
            pltpu.SemaphoreType.DMA,
            pltpu.SemaphoreType.DMA,
        ],
    )(_gather_body)
    return f(user_table, item_table, uid, iid)


NBLK = 2
BB = B // NBLK


def _mlp_body(g_ref, w1_ref, b1_ref, w2_ref, b2_ref, out_ref):
    dn = (((1,), (1,)), ((), ()))
    ht = lax.dot_general(w1_ref[:, :E], g_ref[0, 0], dn,
                         preferred_element_type=jnp.float32)
    ht = ht + lax.dot_general(w1_ref[:, E:], g_ref[1, 0], dn,
                              preferred_element_type=jnp.float32)
    ht = jnp.maximum(ht + b1_ref[...], 0.0)
    out_ref[...] = jnp.sum(ht * w2_ref[...], axis=0) + b2_ref[0, 0]


def _tc_mlp(g, W1, b1, W2, b2):
    g4 = g.reshape(2, NBLK, BB, E)
    return pl.pallas_call(
        _mlp_body,
        grid=(NBLK,),
        in_specs=[
            pl.BlockSpec((2, 1, BB, E), lambda i: (0, i, 0, 0)),
            pl.BlockSpec((H, 2 * E), lambda i: (0, 0)),
            pl.BlockSpec((H, 1), lambda i: (0, 0)),
            pl.BlockSpec((H, 1), lambda i: (0, 0)),
            pl.BlockSpec((1, 1), lambda i: (0, 0)),
        ],
        out_specs=pl.BlockSpec((BB,), lambda i: (i,)),
        out_shape=jax.ShapeDtypeStruct((B,), jnp.float32),
    )(g4, W1, b1.reshape(H, 1), W2.reshape(H, 1), b2.reshape(1, 1))


@jax.jit
def kernel(user_id, item_id, user_table, item_table, W1, b1, W2, b2):
    g = _sc_gather(user_table, item_table,
                   user_id.astype(jnp.int32), item_id.astype(jnp.int32))
    return _tc_mlp(g, W1, b1, W2, b2).reshape(B, 1)


# speedup vs baseline: 1.0223x; 1.0223x over previous
import functools

import jax
import jax.numpy as jnp
from jax import lax
from jax.experimental import pallas as pl
from jax.experimental.pallas import tpu as pltpu
from jax.experimental.pallas import tpu_sc as plsc

B = 4096
E = 128
H = 256
NC = 2
NS = 16
NW = NC * NS
BPW = B // NW
CH = 64


def _gather_body(user_table, item_table, uid, iid, out,
                 uidx, iidx, urows, irows, isem, usem, vsem, wsem):
    wid = lax.axis_index("s") * NC + lax.axis_index("c")
    base = wid * BPW
    ui = pltpu.async_copy(uid.at[pl.ds(base, BPW)], uidx, isem)
    vi = pltpu.async_copy(iid.at[pl.ds(base, BPW)], iidx, isem)
    ui.wait()
    vi.wait()
    gathers = []
    for c in range(0, BPW, CH):
        gathers.append((pltpu.async_copy(user_table.at[uidx.at[pl.ds(c, CH)]],
                                         urows.at[pl.ds(c, CH)], usem),
                        urows.at[pl.ds(c, CH)], base + c))
        gathers.append((pltpu.async_copy(item_table.at[iidx.at[pl.ds(c, CH)]],
                                         irows.at[pl.ds(c, CH)], vsem),
                        irows.at[pl.ds(c, CH)], B + base + c))
    writes = []
    for cp, rows, dst in gathers:
        cp.wait()
        writes.append(pltpu.async_copy(rows, out.at[pl.ds(dst, CH)], wsem))
    for w in writes:
        w.wait()


def _sc_gather(user_table, item_table, uid, iid):
    mesh = plsc.VectorSubcoreMesh(core_axis_name="c", subcore_axis_name="s")
    f = functools.partial(
        pl.kernel,
        mesh=mesh,
        out_type=jax.ShapeDtypeStruct((2 * B, E), jnp.float32),
        scratch_types=[
            pltpu.VMEM((BPW,), jnp.int32),
            pltpu.VMEM((BPW,), jnp.int32),
            pltpu.VMEM((BPW, E), jnp.float32),
            pltpu.VMEM((BPW, E), jnp.float32),
            pltpu.SemaphoreType.DMA,
            pltpu.SemaphoreType.DMA,
            pltpu.SemaphoreType.DMA,
            pltpu.SemaphoreType.DMA,
        ],
    )(_gather_body)
    return f(user_table, item_table, uid, iid)


NBLK = 2
BB = B // NBLK


def _mlp_body(g_ref, w1_ref, b1_ref, w2_ref, b2_ref, out_ref):
    dn = (((1,), (1,)), ((), ()))
    ht = lax.dot_general(w1_ref[:, :E], g_ref[0, 0], dn,
                         preferred_element_type=jnp.float32)
    ht = ht + lax.dot_general(w1_ref[:, E:], g_ref[1, 0], dn,
                              preferred_element_type=jnp.float32)
    ht = jnp.maximum(ht + b1_ref[...], 0.0)
    out_ref[...] = jnp.sum(ht * w2_ref[...], axis=0) + b2_ref[0, 0]


def _tc_mlp(g, W1, b1, W2, b2):
    g4 = g.reshape(2, NBLK, BB, E)
    return pl.pallas_call(
        _mlp_body,
        grid=(NBLK,),
        in_specs=[
            pl.BlockSpec((2, 1, BB, E), lambda i: (0, i, 0, 0)),
            pl.BlockSpec((H, 2 * E), lambda i: (0, 0)),
            pl.BlockSpec((H, 1), lambda i: (0, 0)),
            pl.BlockSpec((H, 1), lambda i: (0, 0)),
            pl.BlockSpec((1, 1), lambda i: (0, 0)),
        ],
        out_specs=pl.BlockSpec((BB,), lambda i: (i,)),
        out_shape=jax.ShapeDtypeStruct((B,), jnp.float32),
    )(g4, W1, b1.reshape(H, 1), W2.reshape(H, 1), b2.reshape(1, 1))


@jax.jit
def kernel(user_id, item_id, user_table, item_table, W1, b1, W2, b2):
    g = _sc_gather(user_table, item_table,
                   user_id.astype(jnp.int32), item_id.astype(jnp.int32))
    return _tc_mlp(g, W1, b1, W2, b2).reshape(B, 1)
